# baseline (speedup 1.0000x reference)
import jax
import jax.numpy as jnp
from jax.experimental import pallas as pl
from jax.experimental.pallas import tpu as pltpu

VOCAB = 100000
D = 64
NCTX = 20
HID = 128
VB = 2048
NVB = (VOCAB + VB - 1) // VB


def _gather_mlp1_kernel(idx_ref, emb_ref, w1_ref, b1_ref, out_ref):
    i = pl.program_id(0)

    @pl.when(i == 0)
    def _():
        out_ref[...] = b1_ref[...]

    out_ref[...] += jnp.dot(emb_ref[0], w1_ref[...],
                            preferred_element_type=jnp.float32)

    @pl.when(i == NCTX - 1)
    def _():
        out_ref[...] = jnp.maximum(out_ref[...], 0.0)


def _mlp2_logsoftmax_kernel(h_ref, w2_ref, b2_ref, out_ref, m_ref, s_ref):
    i = pl.program_id(0)

    @pl.when(i == 0)
    def _():
        m_ref[0] = -jnp.inf
        s_ref[0] = 0.0

    z = jnp.dot(h_ref[...], w2_ref[...],
                preferred_element_type=jnp.float32) + b2_ref[...]
    col = i * VB + jax.lax.broadcasted_iota(jnp.int32, (1, VB), 1)
    z = jnp.where(col < VOCAB, z, -jnp.inf)

    m_old = m_ref[0]
    m_new = jnp.maximum(m_old, jnp.max(z))
    s_ref[0] = s_ref[0] * jnp.exp(m_old - m_new) + jnp.sum(jnp.exp(z - m_new))
    m_ref[0] = m_new

    out_ref[pl.ds(i, 1), :] = z

    @pl.when(i == NVB - 1)
    def _():
        out_ref[...] = out_ref[...] - (m_ref[0] + jnp.log(s_ref[0]))


def kernel(inputs, emb_table, W1, b1, W2, b2):
    idx = inputs.astype(jnp.int32)
    emb3 = emb_table.reshape(VOCAB, 1, D)

    h = pl.pallas_call(
        _gather_mlp1_kernel,
        grid_spec=pltpu.PrefetchScalarGridSpec(
            num_scalar_prefetch=1,
            grid=(NCTX,),
            in_specs=[
                pl.BlockSpec((1, 1, D), lambda i, idx_ref: (idx_ref[i], 0, 0)),
                pl.BlockSpec((D, HID), lambda i, idx_ref: (i, 0)),
                pl.BlockSpec((1, HID), lambda i, idx_ref: (0, 0)),
            ],
            out_specs=pl.BlockSpec((1, HID), lambda i, idx_ref: (0, 0)),
        ),
        out_shape=jax.ShapeDtypeStruct((1, HID), jnp.float32),
    )(idx, emb3, W1, b1.reshape(1, HID))

    logits = pl.pallas_call(
        _mlp2_logsoftmax_kernel,
        grid=(NVB,),
        in_specs=[
            pl.BlockSpec((1, HID), lambda i: (0, 0)),
            pl.BlockSpec((HID, VB), lambda i: (0, i)),
            pl.BlockSpec((1, VB), lambda i: (0, i)),
        ],
        out_specs=pl.BlockSpec((NVB, VB), lambda i: (0, 0)),
        out_shape=jax.ShapeDtypeStruct((NVB, VB), jnp.float32),
        scratch_shapes=[pltpu.SMEM((1,), jnp.float32),
                        pltpu.SMEM((1,), jnp.float32)],
    )(h, W2, b2.reshape(1, VOCAB))

    return logits.reshape(1, NVB * VB)[:, :VOCAB]


# speedup vs baseline: 1.1860x; 1.1860x over previous
import jax
import jax.numpy as jnp
from jax.experimental import pallas as pl
from jax.experimental.pallas import tpu as pltpu

VOCAB = 100000
D = 64
NCTX = 20
HID = 128
VB = 8192
NVB = (VOCAB + VB - 1) // VB


def _gather_mlp1_kernel(idx_ref, emb_ref, w1_ref, b1_ref, out_ref):
    i = pl.program_id(0)

    @pl.when(i == 0)
    def _():
        out_ref[...] = b1_ref[...]

    out_ref[...] += jnp.dot(emb_ref[0], w1_ref[...],
                            preferred_element_type=jnp.float32)

    @pl.when(i == NCTX - 1)
    def _():
        out_ref[...] = jnp.maximum(out_ref[...], 0.0)


def _mlp2_logsoftmax_kernel(h_ref, w2_ref, b2_ref, out_ref,
                            acc_ref, m_ref, s_ref):
    p = pl.program_id(0)
    i = pl.program_id(1)

    @pl.when((p == 0) & (i == 0))
    def _():
        m_ref[0] = -jnp.inf
        s_ref[0] = 0.0

    @pl.when(p == 0)
    def _():
        z = jnp.dot(h_ref[...], w2_ref[...],
                    preferred_element_type=jnp.float32) + b2_ref[...]
        col = i * VB + jax.lax.broadcasted_iota(jnp.int32, (1, VB), 1)
        z = jnp.where(col < VOCAB, z, -jnp.inf)

        m_old = m_ref[0]
        m_new = jnp.maximum(m_old, jnp.max(z))
        s_ref[0] = (s_ref[0] * jnp.exp(m_old - m_new)
                    + jnp.sum(jnp.exp(z - m_new)))
        m_ref[0] = m_new
        acc_ref[pl.ds(i, 1), :] = z

    @pl.when(p == 1)
    def _():
        norm = m_ref[0] + jnp.log(s_ref[0])
        out_ref[...] = acc_ref[pl.ds(i, 1), :] - norm


def kernel(inputs, emb_table, W1, b1, W2, b2):
    idx = inputs.astype(jnp.int32)
    emb3 = emb_table.reshape(VOCAB, 1, D)

    h = pl.pallas_call(
        _gather_mlp1_kernel,
        grid_spec=pltpu.PrefetchScalarGridSpec(
            num_scalar_prefetch=1,
            grid=(NCTX,),
            in_specs=[
                pl.BlockSpec((1, 1, D), lambda i, idx_ref: (idx_ref[i], 0, 0)),
                pl.BlockSpec((D, HID), lambda i, idx_ref: (i, 0)),
                pl.BlockSpec((1, HID), lambda i, idx_ref: (0, 0)),
            ],
            out_specs=pl.BlockSpec((1, HID), lambda i, idx_ref: (0, 0)),
        ),
        out_shape=jax.ShapeDtypeStruct((1, HID), jnp.float32),
    )(idx, emb3, W1, b1.reshape(1, HID))

    log_probs = pl.pallas_call(
        _mlp2_logsoftmax_kernel,
        grid=(2, NVB),
        in_specs=[
            pl.BlockSpec((1, HID), lambda p, i: (0, 0)),
            pl.BlockSpec((HID, VB), lambda p, i: (0, jnp.where(p == 0, i, 0))),
            pl.BlockSpec((1, VB), lambda p, i: (0, jnp.where(p == 0, i, 0))),
        ],
        out_specs=pl.BlockSpec((1, VB),
                               lambda p, i: (0, jnp.where(p == 0, 0, i))),
        out_shape=jax.ShapeDtypeStruct((1, VOCAB), jnp.float32),
        scratch_shapes=[pltpu.VMEM((NVB, VB), jnp.float32),
                        pltpu.SMEM((1,), jnp.float32),
                        pltpu.SMEM((1,), jnp.float32)],
    )(h, W2, b2.reshape(1, VOCAB))

    return log_probs
